# baseline (speedup 1.0000x reference)
import functools

import jax
import jax.numpy as jnp
from jax import lax
from jax.experimental import pallas as pl
from jax.experimental.pallas import tpu as pltpu
from jax.experimental.pallas import tpu_sc as plsc

_D = 128
_QUEUE = 65536
_INV_T = 10.0
_B = 256
_W = 2048
_NBLK = _QUEUE // _W
_NCOL = 1 + 2048 + _QUEUE
_GRID = (_NCOL + _W - 1) // _W


def _out_body(q_ref, k_ref, pos_ref, neg_ref, mem_ref, out_ref, stash_ref):
    t = pl.program_id(0)
    q = q_ref[:]
    prev = stash_ref[:, 0:1]

    l_pos_k = jnp.sum(q * k_ref[:], axis=1, keepdims=True) * _INV_T
    l_pos_set = jnp.mean(
        jnp.sum(q[:, None, :] * pos_ref[:], axis=2), axis=1, keepdims=True
    ) * _INV_T

    is0 = t == 0
    col0_top = jnp.where(is0, l_pos_k, prev)
    col0_bot = jnp.where(is0, l_pos_set, prev)

    src = jnp.where(is0, neg_ref[:], mem_ref[:])
    p = lax.dot_general(
        q, src, (((1,), (1,)), ((), ())), preferred_element_type=jnp.float32
    ) * _INV_T
    rolled = pltpu.roll(p, shift=1, axis=1)

    lane = lax.broadcasted_iota(jnp.int32, (_B, _W), 1)
    out_ref[0:_B, :] = jnp.where(lane == 0, col0_top, rolled)
    out_ref[_B:2 * _B, :] = jnp.where(lane == 0, col0_bot, rolled)
    stash_ref[:, 0:1] = rolled[:, 0:1]


def _logits(q, k, pos_set, neg_flat, memory):
    return pl.pallas_call(
        _out_body,
        grid=(_GRID,),
        in_specs=[
            pl.BlockSpec((_B, _D), lambda t: (0, 0)),
            pl.BlockSpec((_B, _D), lambda t: (0, 0)),
            pl.BlockSpec((_B, 4, _D), lambda t: (0, 0, 0)),
            pl.BlockSpec((_W, _D), lambda t: (0, 0)),
            pl.BlockSpec(
                (_W, _D),
                lambda t: (jnp.clip(t - 1, 0, _NBLK - 1), 0),
            ),
        ],
        out_specs=pl.BlockSpec((2 * _B, _W), lambda t: (0, t)),
        out_shape=jax.ShapeDtypeStruct((2 * _B, _NCOL), jnp.float32),
        scratch_shapes=[pltpu.VMEM((_B, _D), jnp.float32)],
    )(q, k, pos_set, neg_flat, memory)


def _queue_update(memory, k_all):
    info = plsc.get_sparse_core_info()
    nw = info.num_cores * info.num_subcores
    rows_per = _QUEUE // nw
    mesh = plsc.VectorSubcoreMesh(core_axis_name="c", subcore_axis_name="s")

    @functools.partial(
        pl.kernel,
        mesh=mesh,
        out_type=jax.ShapeDtypeStruct((_QUEUE, _D), jnp.float32),
    )
    def body(mem_hbm, kall_hbm, out_hbm):
        wid = lax.axis_index("s") * info.num_cores + lax.axis_index("c")
        base = wid * rows_per

        @pl.when(wid == 0)
        def _():
            pltpu.sync_copy(kall_hbm, out_hbm.at[pl.ds(0, _B)])
            pltpu.sync_copy(
                mem_hbm.at[pl.ds(_B, rows_per - _B)],
                out_hbm.at[pl.ds(_B, rows_per - _B)],
            )

        @pl.when(wid != 0)
        def _():
            pltpu.sync_copy(
                mem_hbm.at[pl.ds(base, rows_per)],
                out_hbm.at[pl.ds(base, rows_per)],
            )

    return body(memory, k_all)


def kernel(q, k, pos_set, neg_set, k_all, memory):
    neg_flat = neg_set.reshape(-1, _D)
    out = _logits(q, k, pos_set, neg_flat, memory)
    new_memory = _queue_update(memory, k_all)
    return (out, new_memory)


# speedup vs baseline: 4.6253x; 4.6253x over previous
import functools

import jax
import jax.numpy as jnp
from jax import lax
from jax.experimental import pallas as pl
from jax.experimental.pallas import tpu as pltpu
from jax.experimental.pallas import tpu_sc as plsc

_D = 128
_QUEUE = 65536
_INV_T = 10.0
_B = 256
_W = 2048
_NBLK = _QUEUE // _W
_NCOL = 1 + 2048 + _QUEUE
_GRID = (_NCOL + _W - 1) // _W


def _out_body(q_ref, k_ref, pos_ref, neg_ref, mem_ref, out_ref, stash_ref):
    t = pl.program_id(0)
    q = q_ref[:]
    prev = stash_ref[:, 0:1]

    l_pos_k = jnp.sum(q * k_ref[:], axis=1, keepdims=True) * _INV_T
    l_pos_set = jnp.mean(
        jnp.sum(q[:, None, :] * pos_ref[:], axis=2), axis=1, keepdims=True
    ) * _INV_T

    is0 = t == 0
    col0_top = jnp.where(is0, l_pos_k, prev)
    col0_bot = jnp.where(is0, l_pos_set, prev)

    src = jnp.where(is0, neg_ref[:], mem_ref[:])
    p = lax.dot_general(
        q, src, (((1,), (1,)), ((), ())), preferred_element_type=jnp.float32
    ) * _INV_T
    rolled = pltpu.roll(p, shift=1, axis=1)

    lane = lax.broadcasted_iota(jnp.int32, (_B, _W), 1)
    out_ref[0:_B, :] = jnp.where(lane == 0, col0_top, rolled)
    out_ref[_B:2 * _B, :] = jnp.where(lane == 0, col0_bot, rolled)
    stash_ref[:, 0:1] = rolled[:, 0:1]


def _logits(q, k, pos_set, neg_flat, memory):
    return pl.pallas_call(
        _out_body,
        grid=(_GRID,),
        in_specs=[
            pl.BlockSpec((_B, _D), lambda t: (0, 0)),
            pl.BlockSpec((_B, _D), lambda t: (0, 0)),
            pl.BlockSpec((_B, 4, _D), lambda t: (0, 0, 0)),
            pl.BlockSpec((_W, _D), lambda t: (0, 0)),
            pl.BlockSpec(
                (_W, _D),
                lambda t: (jnp.clip(t - 1, 0, _NBLK - 1), 0),
            ),
        ],
        out_specs=pl.BlockSpec((2 * _B, _W), lambda t: (0, t)),
        out_shape=jax.ShapeDtypeStruct((2 * _B, _NCOL), jnp.float32),
        scratch_shapes=[pltpu.VMEM((_B, _D), jnp.float32)],
    )(q, k, pos_set, neg_flat, memory)


def _queue_update(memory, k_all):
    info = plsc.get_sparse_core_info()
    nw = info.num_cores * info.num_subcores
    rows_per = _QUEUE // nw
    ch = 128
    nch = rows_per // ch
    mesh = plsc.VectorSubcoreMesh(core_axis_name="c", subcore_axis_name="s")

    @functools.partial(
        pl.kernel,
        mesh=mesh,
        out_type=jax.ShapeDtypeStruct((_QUEUE, _D), jnp.float32),
        scratch_types=[
            pltpu.VMEM((ch, _D), jnp.float32),
            pltpu.VMEM((ch, _D), jnp.float32),
            pltpu.SemaphoreType.DMA,
            pltpu.SemaphoreType.DMA,
            pltpu.SemaphoreType.DMA,
            pltpu.SemaphoreType.DMA,
        ],
    )
    def body(mem_hbm, kall_hbm, out_hbm, buf0, buf1, si0, si1, so0, so1):
        wid = lax.axis_index("s") * info.num_cores + lax.axis_index("c")
        base = wid * rows_per
        bufs = (buf0, buf1)
        sin = (si0, si1)
        sout = (so0, so1)

        def start_in(c):
            b = bufs[c % 2]
            sem = sin[c % 2]
            if c * ch < _B:
                @pl.when(wid == 0)
                def _():
                    pltpu.make_async_copy(
                        kall_hbm.at[pl.ds(c * ch, ch)], b, sem
                    ).start()

                @pl.when(wid != 0)
                def _():
                    pltpu.make_async_copy(
                        mem_hbm.at[pl.ds(base + c * ch, ch)], b, sem
                    ).start()
            else:
                pltpu.make_async_copy(
                    mem_hbm.at[pl.ds(base + c * ch, ch)], b, sem
                ).start()

        start_in(0)
        start_in(1)
        for c in range(nch):
            b = bufs[c % 2]
            pltpu.make_async_copy(
                mem_hbm.at[pl.ds(base + c * ch, ch)], b, sin[c % 2]
            ).wait()
            out_cp = pltpu.make_async_copy(
                b, out_hbm.at[pl.ds(base + c * ch, ch)], sout[c % 2]
            )
            out_cp.start()
            out_cp.wait()
            if c + 2 < nch:
                start_in(c + 2)

    return body(memory, k_all)


def kernel(q, k, pos_set, neg_set, k_all, memory):
    neg_flat = neg_set.reshape(-1, _D)
    out = _logits(q, k, pos_set, neg_flat, memory)
    new_memory = _queue_update(memory, k_all)
    return (out, new_memory)
